# baseline (speedup 1.0000x reference)
import functools

import jax
import jax.numpy as jnp
from jax import lax
from jax.experimental import pallas as pl
from jax.experimental.pallas import tpu as pltpu
from jax.experimental.pallas import tpu_sc as plsc

_NC = 2
_NS = 16
_NW = _NC * _NS
_C = 80
_L = 16


def _scale_body(z_ref, r_ref, o_ref):
    o_ref[...] = z_ref[...] * r_ref[...]


def _prescale(z, rel_emb_weight):
    return pl.pallas_call(
        _scale_body,
        out_shape=jax.ShapeDtypeStruct(z.shape, z.dtype),
    )(z, rel_emb_weight)


def _make_sc_kernel(n_chunk_rows, d):
    ch_per_w = n_chunk_rows // _NW
    mesh = plsc.VectorSubcoreMesh(core_axis_name="c", subcore_axis_name="s")

    @functools.partial(
        pl.kernel,
        mesh=mesh,
        compiler_params=pltpu.CompilerParams(needs_layout_passes=False),
        out_type=jax.ShapeDtypeStruct((_NW, ch_per_w, _C), jnp.float32),
        scratch_types=[
            pltpu.VMEM((ch_per_w, _C), jnp.int32),
            pltpu.VMEM((ch_per_w, _C), jnp.int32),
            pltpu.VMEM((_C, d), jnp.float32),
            pltpu.VMEM((_C, d), jnp.float32),
            pltpu.VMEM((ch_per_w, _C), jnp.float32),
            pltpu.SemaphoreType.DMA,
        ],
    )
    def sck(zr_hbm, z_hbm, h_hbm, t_hbm, out_hbm,
            ih_v, it_v, hr_v, tr_v, out_v, sem):
        wid = lax.axis_index("s") * _NC + lax.axis_index("c")
        pltpu.sync_copy(h_hbm.at[wid], ih_v)
        pltpu.sync_copy(t_hbm.at[wid], it_v)
        iota = lax.iota(jnp.int32, _L)

        n_groups = _C // _L
        rows_g = [iota + (g * _L) for g in range(n_groups)]
        zero = jnp.zeros((_L,), jnp.float32)

        def chunk(k, carry):
            cp_h = pltpu.async_copy(zr_hbm.at[ih_v.at[k]], hr_v, sem)
            cp_t = pltpu.async_copy(z_hbm.at[it_v.at[k]], tr_v, sem)
            cp_h.wait()
            cp_t.wait()

            @plsc.parallel_loop(0, d, 1, unroll=4, carry=(zero,) * n_groups)
            def accs(dd, accs):
                cols = jnp.full((_L,), dd, jnp.int32)
                return tuple(
                    a
                    + plsc.load_gather(hr_v, [rows_g[g], cols])
                    * plsc.load_gather(tr_v, [rows_g[g], cols])
                    for g, a in enumerate(accs)
                )

            for g in range(n_groups):
                out_v[k, pl.ds(g * _L, _L)] = accs[g]
            return carry

        lax.fori_loop(0, ch_per_w, chunk, 0)
        pltpu.sync_copy(out_v, out_hbm.at[wid])

    return sck


def kernel(z, edge_label_index, rel_emb_weight):
    n_nodes, d = z.shape
    n_edges = edge_label_index.shape[1]
    zr = _prescale(z, rel_emb_weight)
    heads = edge_label_index[0].reshape(_NW, -1, _C)
    tails = edge_label_index[1].reshape(_NW, -1, _C)
    sck = _make_sc_kernel(_NW * heads.shape[1], d)
    out2 = sck(zr, z, heads, tails)
    return out2.reshape(n_edges)


# speedup vs baseline: 4.8843x; 4.8843x over previous
import functools

import jax
import jax.numpy as jnp
from jax import lax
from jax.experimental import pallas as pl
from jax.experimental.pallas import tpu as pltpu
from jax.experimental.pallas import tpu_sc as plsc

_NC = 2
_NS = 16
_NW = _NC * _NS
_C = 80
_L = 16


def _scale_body(z_ref, r_ref, o_ref):
    o_ref[...] = z_ref[...] * r_ref[...]


def _prescale(z, rel_emb_weight):
    return pl.pallas_call(
        _scale_body,
        out_shape=jax.ShapeDtypeStruct(z.shape, z.dtype),
    )(z, rel_emb_weight)


def _make_sc_kernel(n_chunk_rows, d):
    ch_per_w = n_chunk_rows // _NW
    mesh = plsc.VectorSubcoreMesh(core_axis_name="c", subcore_axis_name="s")

    @functools.partial(
        pl.kernel,
        mesh=mesh,
        compiler_params=pltpu.CompilerParams(needs_layout_passes=False),
        out_type=jax.ShapeDtypeStruct((_NW, ch_per_w, _C), jnp.float32),
        scratch_types=[
            pltpu.VMEM((ch_per_w, _C), jnp.int32),
            pltpu.VMEM((ch_per_w, _C), jnp.int32),
            pltpu.VMEM((_C, d), jnp.float32),
            pltpu.VMEM((_C, d), jnp.float32),
            pltpu.VMEM((ch_per_w, _C), jnp.float32),
            pltpu.SemaphoreType.DMA,
        ],
    )
    def sck(zr_hbm, z_hbm, h_hbm, t_hbm, out_hbm,
            ih_v, it_v, hr_v, tr_v, out_v, sem):
        wid = lax.axis_index("s") * _NC + lax.axis_index("c")
        pltpu.sync_copy(h_hbm.at[wid], ih_v)
        pltpu.sync_copy(t_hbm.at[wid], it_v)
        iota = lax.iota(jnp.int32, _L)

        n_groups = _C // _L
        rows_g = [iota + (g * _L) for g in range(n_groups)]
        zero = jnp.zeros((_L,), jnp.float32)

        def chunk(k, carry):
            cp_h = pltpu.async_copy(zr_hbm.at[ih_v.at[k]], hr_v, sem)
            cp_t = pltpu.async_copy(z_hbm.at[it_v.at[k]], tr_v, sem)
            cp_h.wait()
            cp_t.wait()

            @plsc.parallel_loop(0, d, 1, unroll=4, carry=(zero,) * n_groups)
            def accs(dd, accs):
                cols = ((iota + dd) & 15) + (dd & -16)
                return tuple(
                    a
                    + plsc.load_gather(hr_v, [rows_g[g], cols])
                    * plsc.load_gather(tr_v, [rows_g[g], cols])
                    for g, a in enumerate(accs)
                )

            for g in range(n_groups):
                out_v[k, pl.ds(g * _L, _L)] = accs[g]
            return carry

        lax.fori_loop(0, ch_per_w, chunk, 0)
        pltpu.sync_copy(out_v, out_hbm.at[wid])

    return sck


def kernel(z, edge_label_index, rel_emb_weight):
    n_nodes, d = z.shape
    n_edges = edge_label_index.shape[1]
    zr = _prescale(z, rel_emb_weight)
    heads = edge_label_index[0].reshape(_NW, -1, _C)
    tails = edge_label_index[1].reshape(_NW, -1, _C)
    sck = _make_sc_kernel(_NW * heads.shape[1], d)
    out2 = sck(zr, z, heads, tails)
    return out2.reshape(n_edges)


# speedup vs baseline: 8.1390x; 1.6664x over previous
import functools

import jax
import jax.numpy as jnp
from jax import lax
from jax.experimental import pallas as pl
from jax.experimental.pallas import tpu as pltpu
from jax.experimental.pallas import tpu_sc as plsc

_NC = 2
_NS = 16
_NW = _NC * _NS
_C = 80
_L = 16


def _scale_body(z_ref, r_ref, o_ref):
    o_ref[...] = z_ref[...] * r_ref[...]


def _prescale(z, rel_emb_weight):
    return pl.pallas_call(
        _scale_body,
        out_shape=jax.ShapeDtypeStruct(z.shape, z.dtype),
    )(z, rel_emb_weight)


def _make_sc_kernel(n_chunk_rows, d):
    ch_per_w = n_chunk_rows // _NW
    mesh = plsc.VectorSubcoreMesh(core_axis_name="c", subcore_axis_name="s")

    @functools.partial(
        pl.kernel,
        mesh=mesh,
        compiler_params=pltpu.CompilerParams(needs_layout_passes=False),
        out_type=jax.ShapeDtypeStruct((_NW, ch_per_w, _C), jnp.float32),
        scratch_types=[
            pltpu.VMEM((ch_per_w, _C), jnp.int32),
            pltpu.VMEM((ch_per_w, _C), jnp.int32),
            pltpu.VMEM((2, _C, d), jnp.float32),
            pltpu.VMEM((2, _C, d), jnp.float32),
            pltpu.VMEM((ch_per_w, _C), jnp.float32),
            pltpu.SemaphoreType.DMA((2,)),
        ],
    )
    def sck(zr_hbm, z_hbm, h_hbm, t_hbm, out_hbm,
            ih_v, it_v, hr_v, tr_v, out_v, sem):
        wid = lax.axis_index("s") * _NC + lax.axis_index("c")
        pltpu.sync_copy(h_hbm.at[wid], ih_v)
        pltpu.sync_copy(t_hbm.at[wid], it_v)
        iota = lax.iota(jnp.int32, _L)

        n_groups = _C // _L
        rows_g = [iota + (g * _L) for g in range(n_groups)]
        zero = jnp.zeros((_L,), jnp.float32)

        pltpu.async_copy(zr_hbm.at[ih_v.at[0]], hr_v.at[0], sem.at[0])
        pltpu.async_copy(z_hbm.at[it_v.at[0]], tr_v.at[0], sem.at[0])

        def chunk(k, carry):
            b = lax.rem(k, 2)
            nb = 1 - b

            @pl.when(k + 1 < ch_per_w)
            def _prefetch():
                pltpu.async_copy(zr_hbm.at[ih_v.at[k + 1]], hr_v.at[nb],
                                 sem.at[nb])
                pltpu.async_copy(z_hbm.at[it_v.at[k + 1]], tr_v.at[nb],
                                 sem.at[nb])

            pltpu.make_async_copy(zr_hbm.at[ih_v.at[k]], hr_v.at[b],
                                  sem.at[b]).wait()
            pltpu.make_async_copy(z_hbm.at[it_v.at[k]], tr_v.at[b],
                                  sem.at[b]).wait()
            hr_b = hr_v.at[b]
            tr_b = tr_v.at[b]

            @plsc.parallel_loop(0, d, 1, unroll=4, carry=(zero,) * n_groups)
            def accs(dd, accs):
                cols = ((iota + dd) & 15) + (dd & -16)
                return tuple(
                    a
                    + plsc.load_gather(hr_b, [rows_g[g], cols])
                    * plsc.load_gather(tr_b, [rows_g[g], cols])
                    for g, a in enumerate(accs)
                )

            for g in range(n_groups):
                out_v[k, pl.ds(g * _L, _L)] = accs[g]
            return carry

        lax.fori_loop(0, ch_per_w, chunk, 0)
        pltpu.sync_copy(out_v, out_hbm.at[wid])

    return sck


def kernel(z, edge_label_index, rel_emb_weight):
    n_nodes, d = z.shape
    n_edges = edge_label_index.shape[1]
    zr = _prescale(z, rel_emb_weight)
    heads = edge_label_index[0].reshape(_NW, -1, _C)
    tails = edge_label_index[1].reshape(_NW, -1, _C)
    sck = _make_sc_kernel(_NW * heads.shape[1], d)
    out2 = sck(zr, z, heads, tails)
    return out2.reshape(n_edges)


# speedup vs baseline: 9.8440x; 1.2095x over previous
import functools

import jax
import jax.numpy as jnp
from jax import lax
from jax.experimental import pallas as pl
from jax.experimental.pallas import tpu as pltpu
from jax.experimental.pallas import tpu_sc as plsc

_NC = 2
_NS = 16
_NW = _NC * _NS
_C = 80
_L = 16
_NBUF = 4


def _scale_body(z_ref, r_ref, o_ref):
    o_ref[...] = z_ref[...] * r_ref[...]


def _prescale(z, rel_emb_weight):
    return pl.pallas_call(
        _scale_body,
        out_shape=jax.ShapeDtypeStruct(z.shape, z.dtype),
    )(z, rel_emb_weight)


def _make_sc_kernel(n_chunk_rows, d):
    ch_per_w = n_chunk_rows // _NW
    mesh = plsc.VectorSubcoreMesh(core_axis_name="c", subcore_axis_name="s")

    @functools.partial(
        pl.kernel,
        mesh=mesh,
        compiler_params=pltpu.CompilerParams(needs_layout_passes=False),
        out_type=jax.ShapeDtypeStruct((_NW, ch_per_w, _C), jnp.float32),
        scratch_types=[
            pltpu.VMEM((ch_per_w, _C), jnp.int32),
            pltpu.VMEM((ch_per_w, _C), jnp.int32),
            pltpu.VMEM((_NBUF, _C, d), jnp.float32),
            pltpu.VMEM((_NBUF, _C, d), jnp.float32),
            pltpu.VMEM((ch_per_w, _C), jnp.float32),
            pltpu.SemaphoreType.DMA((_NBUF,)),
        ],
    )
    def sck(zr_hbm, z_hbm, h_hbm, t_hbm, out_hbm,
            ih_v, it_v, hr_v, tr_v, out_v, sem):
        wid = lax.axis_index("s") * _NC + lax.axis_index("c")
        pltpu.sync_copy(h_hbm.at[wid], ih_v)
        pltpu.sync_copy(t_hbm.at[wid], it_v)
        iota = lax.iota(jnp.int32, _L)

        n_groups = _C // _L
        rows_g = [iota + (g * _L) for g in range(n_groups)]
        zero = jnp.zeros((_L,), jnp.float32)

        for j in range(_NBUF - 1):
            pltpu.async_copy(zr_hbm.at[ih_v.at[j]], hr_v.at[j], sem.at[j])
            pltpu.async_copy(z_hbm.at[it_v.at[j]], tr_v.at[j], sem.at[j])

        def chunk(k, carry):
            b = lax.rem(k, _NBUF)
            kn = k + _NBUF - 1
            nb = lax.rem(kn, _NBUF)

            @pl.when(kn < ch_per_w)
            def _prefetch():
                pltpu.async_copy(zr_hbm.at[ih_v.at[kn]], hr_v.at[nb],
                                 sem.at[nb])
                pltpu.async_copy(z_hbm.at[it_v.at[kn]], tr_v.at[nb],
                                 sem.at[nb])

            pltpu.make_async_copy(zr_hbm.at[ih_v.at[k]], hr_v.at[b],
                                  sem.at[b]).wait()
            pltpu.make_async_copy(z_hbm.at[it_v.at[k]], tr_v.at[b],
                                  sem.at[b]).wait()
            hr_b = hr_v.at[b]
            tr_b = tr_v.at[b]

            @plsc.parallel_loop(0, d, 1, unroll=4, carry=(zero,) * n_groups)
            def accs(dd, accs):
                cols = ((iota + dd) & 15) + (dd & -16)
                return tuple(
                    a
                    + plsc.load_gather(hr_b, [rows_g[g], cols])
                    * plsc.load_gather(tr_b, [rows_g[g], cols])
                    for g, a in enumerate(accs)
                )

            for g in range(n_groups):
                out_v[k, pl.ds(g * _L, _L)] = accs[g]
            return carry

        lax.fori_loop(0, ch_per_w, chunk, 0)
        pltpu.sync_copy(out_v, out_hbm.at[wid])

    return sck


def kernel(z, edge_label_index, rel_emb_weight):
    n_nodes, d = z.shape
    n_edges = edge_label_index.shape[1]
    zr = _prescale(z, rel_emb_weight)
    heads = edge_label_index[0].reshape(_NW, -1, _C)
    tails = edge_label_index[1].reshape(_NW, -1, _C)
    sck = _make_sc_kernel(_NW * heads.shape[1], d)
    out2 = sck(zr, z, heads, tails)
    return out2.reshape(n_edges)
